# baseline (speedup 1.0000x reference)
import functools

import jax
import jax.numpy as jnp
from jax import lax
from jax.experimental import pallas as pl
from jax.experimental.pallas import tpu as pltpu
from jax.experimental.pallas import tpu_sc as plsc

N = 10000
E = 320000
D = 128
DE = 16
HE = 16
OUT = 128

NC = 2
NS = 16
NW = NC * NS

NH = 2
EPT = E // NH // NW
CHUNK = 40
NCH = EPT // CHUNK
NP = 10240
ROWS_PER_SUB = NP // NS


def _gelu(h):
    return 0.5 * h * (1.0 + lax.erf(h * (2.0 ** -0.5)))


E8 = E // 8
EMLP_BLK = 2000


def _emlp_body(ef_ref, w1_ref, b1_ref, w2_ref, b2_ref, wp_ref, bp_ref, o_ref):
    h = jnp.dot(ef_ref[...], w1_ref[...], preferred_element_type=jnp.float32)
    h = _gelu(h + b1_ref[...])
    ep = jnp.dot(h.astype(jnp.bfloat16), w2_ref[...],
                 preferred_element_type=jnp.float32) + b2_ref[...]
    y = (jnp.dot(ep.astype(jnp.bfloat16), wp_ref[...],
                 preferred_element_type=jnp.float32)
         + bp_ref[...])
    o_ref[...] = y.reshape(EMLP_BLK * 8, 128)


def _edge_mlp(ef8, w1b, b1t, w2b, b2t, wpb, bpt, half):
    nblk = E8 // EMLP_BLK // NH
    off = half * nblk
    return pl.pallas_call(
        _emlp_body,
        grid=(nblk,),
        in_specs=[
            pl.BlockSpec((EMLP_BLK, 128), lambda i: (i + off, 0)),
            pl.BlockSpec((128, 128), lambda i: (0, 0)),
            pl.BlockSpec((1, 128), lambda i: (0, 0)),
            pl.BlockSpec((128, 128), lambda i: (0, 0)),
            pl.BlockSpec((1, 128), lambda i: (0, 0)),
            pl.BlockSpec((128, 1024), lambda i: (0, 0)),
            pl.BlockSpec((1, 1024), lambda i: (0, 0)),
        ],
        out_specs=pl.BlockSpec((EMLP_BLK * 8, 128), lambda i: (i, 0)),
        out_shape=jax.ShapeDtypeStruct((E // NH, 128), jnp.float32),
    )(ef8, w1b, b1t, w2b, b2t, wpb, bpt)


def _xw_body(x_ref, wm1x_ref, o_ref):
    o_ref[...] = lax.dot_general(x_ref[...], wm1x_ref[...],
                                 (((1,), (1,)), ((), ())),
                                 preferred_element_type=jnp.float32)


def _xw(x, Wm1x):
    return pl.pallas_call(
        _xw_body,
        out_shape=jax.ShapeDtypeStruct((N, 128), jnp.float32),
    )(x, Wm1x)


NSEG = 5
G = NCH // NSEG


def _sc_body(
    xw_hbm, ei6_hbm, ye_hbm, zx_hbm,
    acc_out,
    acc_sh, src_t, dst_t, rows0, rows1, yrows0, yrows1,
    semg0, semg1, semy0, semy1,
    half,
):
    c = lax.axis_index("c")
    s = lax.axis_index("s")
    wid = c * NS + s
    ebase = wid * EPT

    rbase = s * ROWS_PER_SUB

    def zstep(j, carry):
        rb = rbase + j * CHUNK
        pltpu.sync_copy(zx_hbm.at[pl.ds(rb, CHUNK)], rows0)
        pltpu.sync_copy(rows0, acc_sh.at[pl.ds(rb, CHUNK)])
        return carry

    lax.fori_loop(0, ROWS_PER_SUB // CHUNK, zstep, 0)
    plsc.subcore_barrier()

    def make_ops(sg):
        sbase = ebase + sg * G * CHUNK

        def start(i, rows_b, yrows_b, semg, semy):
            pltpu.async_copy(xw_hbm.at[src_t.at[i]], rows_b, semg)
            pltpu.async_copy(ye_hbm.at[pl.ds(sbase + i * CHUNK, CHUNK)],
                             yrows_b, semy)

        def finish(i, rows_b, yrows_b, semg, semy):
            pltpu.make_async_copy(xw_hbm.at[src_t.at[i]], rows_b, semg).wait()
            pltpu.make_async_copy(
                ye_hbm.at[pl.ds(sbase + i * CHUNK, CHUNK)], yrows_b, semy
            ).wait()
            sc2 = pltpu.async_copy(yrows_b, acc_sh.at[dst_t.at[i]], semy,
                                   add=True)
            pltpu.sync_copy(rows_b, acc_sh.at[dst_t.at[i]], add=True)
            sc2.wait()

        return start, finish

    for sg in range(NSEG):
        pltpu.sync_copy(ei6_hbm.at[0, half, wid, sg], src_t)
        pltpu.sync_copy(ei6_hbm.at[1, half, wid, sg], dst_t)
        start, finish = make_ops(sg)
        start(0, rows0, yrows0, semg0, semy0)

        def pair(j, carry, start=start, finish=finish):
            i0 = 2 * j
            start(i0 + 1, rows1, yrows1, semg1, semy1)
            finish(i0, rows0, yrows0, semg0, semy0)
            start(i0 + 2, rows0, yrows0, semg0, semy0)
            finish(i0 + 1, rows1, yrows1, semg1, semy1)
            return carry

        lax.fori_loop(0, (G - 1) // 2, pair, 0)
        finish(G - 1, rows0, yrows0, semg0, semy0)

    plsc.subcore_barrier()

    obase = c * NP + rbase

    def wstep(j, carry):
        rb = rbase + j * CHUNK
        ob = obase + j * CHUNK
        pltpu.sync_copy(acc_sh.at[pl.ds(rb, CHUNK)], rows0)
        pltpu.sync_copy(rows0, acc_out.at[pl.ds(ob, CHUNK)])
        return carry

    lax.fori_loop(0, ROWS_PER_SUB // CHUNK, wstep, 0)


def _sc_scatter(xw, ei6, ye, zx, half):
    body = functools.partial(_sc_body, half=half)
    mesh = plsc.VectorSubcoreMesh(core_axis_name="c", subcore_axis_name="s")
    f = functools.partial(
        pl.kernel,
        out_type=jax.ShapeDtypeStruct((NC * NP, 128), jnp.float32),
        mesh=mesh,
        scratch_types=[
            pltpu.VMEM_SHARED((NP, 128), jnp.float32),
            pltpu.VMEM((G, CHUNK), jnp.int32),
            pltpu.VMEM((G, CHUNK), jnp.int32),
            pltpu.VMEM((CHUNK, 128), jnp.float32),
            pltpu.VMEM((CHUNK, 128), jnp.float32),
            pltpu.VMEM((CHUNK, 128), jnp.float32),
            pltpu.VMEM((CHUNK, 128), jnp.float32),
            pltpu.SemaphoreType.DMA,
            pltpu.SemaphoreType.DMA,
            pltpu.SemaphoreType.DMA,
            pltpu.SemaphoreType.DMA,
        ],
    )(body)
    return f(xw, ei6, ye, zx)


def _tail_body(x_ref, acca_ref, accb_ref, wego_ref, eps_ref, wm1_ref,
               bm1_ref, g_ref, bb_ref, wm2_ref, bm2_ref, o_ref):
    x = x_ref[...]
    acc = (acca_ref[:N, :] + acca_ref[NP:NP + N, :]
           + accb_ref[:N, :] + accb_ref[NP:NP + N, :])
    wcombo = jnp.dot(wm1_ref[...], wego_ref[...],
                     preferred_element_type=jnp.float32)
    scale = 1.0 + eps_ref[0, 0]
    h1 = (
        scale * lax.dot_general(x, wcombo, (((1,), (1,)), ((), ())),
                                preferred_element_type=jnp.float32)
        + acc
        + bm1_ref[...]
    )
    mean = jnp.mean(h1, axis=0, keepdims=True)
    var = jnp.mean((h1 - mean) ** 2, axis=0, keepdims=True)
    h1 = (h1 - mean) / jnp.sqrt(var + 1e-5) * g_ref[...] + bb_ref[...]
    h1 = _gelu(h1)
    o_ref[...] = (
        lax.dot_general(h1, wm2_ref[...], (((1,), (1,)), ((), ())),
                        preferred_element_type=jnp.float32)
        + bm2_ref[...]
    )


def _tail(x, acca, accb, Wego, eps, Wm1, bm1, bn_g, bn_b, Wm2, bm2):
    return pl.pallas_call(
        _tail_body,
        out_shape=jax.ShapeDtypeStruct((N, OUT), jnp.float32),
    )(x, acca, accb, Wego, eps, Wm1, bm1, bn_g, bn_b, Wm2, bm2)


def kernel(x, edge_idx, edge_feat, We1, be1, We2, be2, Wego, eps, Wm1, bm1,
           bn_g, bn_b, Wm2, bm2):
    ei6 = edge_idx.reshape(2, NH, NW, NSEG, G, CHUNK)

    Wm1x = Wm1[:, :D]
    Wm1e = Wm1[:, D:]

    eye8 = jnp.eye(8, dtype=jnp.float32)
    w1b = jnp.kron(eye8, We1.T)
    w2b = jnp.kron(eye8, We2.T)
    b1t = jnp.tile(be1, 8).reshape(1, 128)
    b2t = jnp.tile(be2, 8).reshape(1, 128)
    wpb = jnp.kron(eye8, Wm1e.T)
    bpt = jnp.zeros((1, 1024), jnp.float32)

    ef8 = edge_feat.reshape(E8, 128).astype(jnp.bfloat16)
    w1c = w1b.astype(jnp.bfloat16)
    w2c = w2b.astype(jnp.bfloat16)
    wpc = wpb.astype(jnp.bfloat16)
    xw = _xw(x, Wm1x)
    zx = jnp.zeros((NP, 128), jnp.float32)

    ye_a = _edge_mlp(ef8, w1c, b1t, w2c, b2t, wpc, bpt, 0)
    acc_a = _sc_scatter(xw, ei6, ye_a, zx, 0)
    ye_b = _edge_mlp(ef8, w1c, b1t, w2c, b2t, wpc, bpt, 1)
    acc_b = _sc_scatter(xw, ei6, ye_b, zx, 1)

    return _tail(x, acc_a, acc_b, Wego, eps.reshape(1, 1), Wm1,
                 bm1.reshape(1, OUT), bn_g.reshape(1, OUT),
                 bn_b.reshape(1, OUT), Wm2, bm2.reshape(1, OUT))


# speedup vs baseline: 1.1118x; 1.1118x over previous
import functools

import jax
import jax.numpy as jnp
from jax import lax
from jax.experimental import pallas as pl
from jax.experimental.pallas import tpu as pltpu
from jax.experimental.pallas import tpu_sc as plsc

N = 10000
E = 320000
D = 128
DE = 16
HE = 16
OUT = 128

NC = 2
NS = 16
NW = NC * NS

EPT = E // NW
CHUNK = 80
NCH = EPT // CHUNK
NP = 10240
ROWS_PER_SUB = NP // NS


def _gelu(h):
    return 0.5 * h * (1.0 + lax.erf(h * (2.0 ** -0.5)))


E8 = E // 8
EMLP_BLK = 2000


def _emlp_body(ef_ref, w1_ref, b1_ref, w2_ref, b2_ref, wp_ref, bp_ref, o_ref):
    h = jnp.dot(ef_ref[...], w1_ref[...], preferred_element_type=jnp.float32)
    h = _gelu(h + b1_ref[...])
    ep = jnp.dot(h.astype(jnp.bfloat16), w2_ref[...],
                 preferred_element_type=jnp.float32) + b2_ref[...]
    y = (jnp.dot(ep.astype(jnp.bfloat16), wp_ref[...],
                 preferred_element_type=jnp.float32)
         + bp_ref[...])
    o_ref[...] = y.reshape(EMLP_BLK * 8, 128)


def _edge_mlp(ef8, w1b, b1t, w2b, b2t, wpb, bpt):
    return pl.pallas_call(
        _emlp_body,
        grid=(E8 // EMLP_BLK,),
        in_specs=[
            pl.BlockSpec((EMLP_BLK, 128), lambda i: (i, 0)),
            pl.BlockSpec((128, 128), lambda i: (0, 0)),
            pl.BlockSpec((1, 128), lambda i: (0, 0)),
            pl.BlockSpec((128, 128), lambda i: (0, 0)),
            pl.BlockSpec((1, 128), lambda i: (0, 0)),
            pl.BlockSpec((128, 1024), lambda i: (0, 0)),
            pl.BlockSpec((1, 1024), lambda i: (0, 0)),
        ],
        out_specs=pl.BlockSpec((EMLP_BLK * 8, 128), lambda i: (i, 0)),
        out_shape=jax.ShapeDtypeStruct((E, 128), jnp.float32),
    )(ef8, w1b, b1t, w2b, b2t, wpb, bpt)


def _xw_body(x_ref, wm1x_ref, o_ref):
    o_ref[...] = lax.dot_general(x_ref[...], wm1x_ref[...],
                                 (((1,), (1,)), ((), ())),
                                 preferred_element_type=jnp.float32)


def _xw(x, Wm1x):
    return pl.pallas_call(
        _xw_body,
        out_shape=jax.ShapeDtypeStruct((N, 128), jnp.float32),
    )(x, Wm1x)


NSEG = 5
G = NCH // NSEG


def _sc_body(
    xw_hbm, ei5_hbm, ye_hbm, zx_hbm,
    acc_out,
    acc_sh, src_t, dst_t, rows0, rows1, yrows0, yrows1,
    semg0, semg1, semy0, semy1,
):
    c = lax.axis_index("c")
    s = lax.axis_index("s")
    wid = c * NS + s
    ebase = wid * EPT

    rbase = s * ROWS_PER_SUB

    def zstep(j, carry):
        rb = rbase + j * CHUNK
        pltpu.sync_copy(zx_hbm.at[pl.ds(rb, CHUNK)], rows0)
        pltpu.sync_copy(rows0, acc_sh.at[pl.ds(rb, CHUNK)])
        return carry

    lax.fori_loop(0, ROWS_PER_SUB // CHUNK, zstep, 0)
    plsc.subcore_barrier()

    def make_ops(sg):
        sbase = ebase + sg * G * CHUNK

        def start(i, rows_b, yrows_b, semg, semy):
            pltpu.async_copy(xw_hbm.at[src_t.at[i]], rows_b, semg)
            pltpu.async_copy(ye_hbm.at[pl.ds(sbase + i * CHUNK, CHUNK)],
                             yrows_b, semy)

        def finish(i, rows_b, yrows_b, semg, semy):
            pltpu.make_async_copy(xw_hbm.at[src_t.at[i]], rows_b, semg).wait()
            pltpu.make_async_copy(
                ye_hbm.at[pl.ds(sbase + i * CHUNK, CHUNK)], yrows_b, semy
            ).wait()
            sc2 = pltpu.async_copy(yrows_b, acc_sh.at[dst_t.at[i]], semy,
                                   add=True)
            pltpu.sync_copy(rows_b, acc_sh.at[dst_t.at[i]], add=True)
            sc2.wait()

        return start, finish

    for sg in range(NSEG):
        pltpu.sync_copy(ei5_hbm.at[0, wid, sg], src_t)
        pltpu.sync_copy(ei5_hbm.at[1, wid, sg], dst_t)
        start, finish = make_ops(sg)
        start(0, rows0, yrows0, semg0, semy0)

        def pair(j, carry, start=start, finish=finish):
            i0 = 2 * j
            start(i0 + 1, rows1, yrows1, semg1, semy1)
            finish(i0, rows0, yrows0, semg0, semy0)
            start(i0 + 2, rows0, yrows0, semg0, semy0)
            finish(i0 + 1, rows1, yrows1, semg1, semy1)
            return carry

        lax.fori_loop(0, (G - 1) // 2, pair, 0)
        finish(G - 1, rows0, yrows0, semg0, semy0)

    plsc.subcore_barrier()

    obase = c * NP + rbase

    def wstep(j, carry):
        rb = rbase + j * CHUNK
        ob = obase + j * CHUNK
        pltpu.sync_copy(acc_sh.at[pl.ds(rb, CHUNK)], rows0)
        pltpu.sync_copy(rows0, acc_out.at[pl.ds(ob, CHUNK)])
        return carry

    lax.fori_loop(0, ROWS_PER_SUB // CHUNK, wstep, 0)


def _sc_scatter(xw, ei5, ye, zx):
    mesh = plsc.VectorSubcoreMesh(core_axis_name="c", subcore_axis_name="s")
    f = functools.partial(
        pl.kernel,
        out_type=jax.ShapeDtypeStruct((NC * NP, 128), jnp.float32),
        mesh=mesh,
        scratch_types=[
            pltpu.VMEM_SHARED((NP, 128), jnp.float32),
            pltpu.VMEM((G, CHUNK), jnp.int32),
            pltpu.VMEM((G, CHUNK), jnp.int32),
            pltpu.VMEM((CHUNK, 128), jnp.float32),
            pltpu.VMEM((CHUNK, 128), jnp.float32),
            pltpu.VMEM((CHUNK, 128), jnp.float32),
            pltpu.VMEM((CHUNK, 128), jnp.float32),
            pltpu.SemaphoreType.DMA,
            pltpu.SemaphoreType.DMA,
            pltpu.SemaphoreType.DMA,
            pltpu.SemaphoreType.DMA,
        ],
    )(_sc_body)
    return f(xw, ei5, ye, zx)


def _tail_body(x_ref, acc_ref, wego_ref, eps_ref, wm1_ref, bm1_ref,
               g_ref, bb_ref, wm2_ref, bm2_ref, o_ref):
    x = x_ref[...]
    acc = acc_ref[:N, :] + acc_ref[NP:NP + N, :]
    wcombo = jnp.dot(wm1_ref[...], wego_ref[...],
                     preferred_element_type=jnp.float32)
    scale = 1.0 + eps_ref[0, 0]
    h1 = (
        scale * lax.dot_general(x, wcombo, (((1,), (1,)), ((), ())),
                                preferred_element_type=jnp.float32)
        + acc
        + bm1_ref[...]
    )
    mean = jnp.mean(h1, axis=0, keepdims=True)
    var = jnp.mean((h1 - mean) ** 2, axis=0, keepdims=True)
    h1 = (h1 - mean) / jnp.sqrt(var + 1e-5) * g_ref[...] + bb_ref[...]
    h1 = _gelu(h1)
    o_ref[...] = (
        lax.dot_general(h1, wm2_ref[...], (((1,), (1,)), ((), ())),
                        preferred_element_type=jnp.float32)
        + bm2_ref[...]
    )


def _tail(x, acc, Wego, eps, Wm1, bm1, bn_g, bn_b, Wm2, bm2):
    return pl.pallas_call(
        _tail_body,
        out_shape=jax.ShapeDtypeStruct((N, OUT), jnp.float32),
    )(x, acc, Wego, eps, Wm1, bm1, bn_g, bn_b, Wm2, bm2)


def kernel(x, edge_idx, edge_feat, We1, be1, We2, be2, Wego, eps, Wm1, bm1,
           bn_g, bn_b, Wm2, bm2):
    ei5 = edge_idx.reshape(2, NW, NSEG, G, CHUNK)

    Wm1x = Wm1[:, :D]
    Wm1e = Wm1[:, D:]

    eye8 = jnp.eye(8, dtype=jnp.float32)
    w1b = jnp.kron(eye8, We1.T)
    w2b = jnp.kron(eye8, We2.T)
    b1t = jnp.tile(be1, 8).reshape(1, 128)
    b2t = jnp.tile(be2, 8).reshape(1, 128)
    wpb = jnp.kron(eye8, Wm1e.T)
    bpt = jnp.zeros((1, 1024), jnp.float32)

    ef8 = edge_feat.reshape(E8, 128).astype(jnp.bfloat16)
    ye = _edge_mlp(ef8, w1b.astype(jnp.bfloat16), b1t,
                   w2b.astype(jnp.bfloat16), b2t,
                   wpb.astype(jnp.bfloat16), bpt)
    xw = _xw(x, Wm1x)

    zx = jnp.zeros((NP, 128), jnp.float32)
    acc = _sc_scatter(xw, ei5, ye, zx)

    return _tail(x, acc, Wego, eps.reshape(1, 1), Wm1,
                 bm1.reshape(1, OUT), bn_g.reshape(1, OUT),
                 bn_b.reshape(1, OUT), Wm2, bm2.reshape(1, OUT))
